# baseline (device time: 31371 ns/iter reference)
import os

import jax
import jax.numpy as jnp
from jax import lax
from jax.experimental import pallas as pl
from jax.experimental.pallas import tpu as pltpu

N_DEV = 8
_INTERPRET = os.environ.get("KERNEL_INTERPRET", "0") == "1"
_MM_DTYPE = jnp.bfloat16 if os.environ.get("KERNEL_MM_BF16") == "1" else jnp.float8_e4m3fn
_ABLATE = os.environ.get("KERNEL_ABLATE", "")


def _mm(a, b):
    return lax.dot_general(
        a, b, (((1,), (0,)), ((), ())), preferred_element_type=jnp.float32
    )


def kernel(x, w_mat, scale_x, scale_w):
    m_glob, k_loc = x.shape
    k_glob, n_out = w_mat.shape
    mb = m_glob // N_DEV
    n_chunk = 2
    spc = N_DEV // n_chunk
    do_comm = _ABLATE != "nocomm"

    def body(x_hbm, w_hbm, acc_hbm,
             xf_ref, xb_ref, xg_ref, wf_ref, wg_ref, acc_ref,
             send_sems, recv_sems, wdma_sems, xdma_sems, odma_sem):
        me = lax.axis_index("i")

        def kblk(p):
            return (me - p) % N_DEV

        def start_wdma(step, slot):
            cp = pltpu.make_async_copy(
                w_hbm.at[pl.ds(kblk(step) * k_loc, k_loc), :],
                wf_ref.at[slot],
                wdma_sems.at[slot],
            )
            cp.start()
            return cp

        def start_xdma(i, slot):
            cp = pltpu.make_async_copy(
                x_hbm.at[pl.ds(((me + i + 1) % N_DEV) * mb, mb), :],
                xf_ref.at[slot],
                xdma_sems.at[slot],
            )
            cp.start()
            return cp

        xcp = [start_xdma(i, i) for i in range(4)]
        wcp = [start_wdma(p, p) for p in range(4)]

        if do_comm:
            barrier = pltpu.get_barrier_semaphore()
            for d in range(1, N_DEV):
                pl.semaphore_signal(
                    barrier, inc=1,
                    device_id=((me + d) % N_DEV,),
                    device_id_type=pl.DeviceIdType.MESH,
                )
            pl.semaphore_wait(barrier, N_DEV - 1)

        rdmas = []
        for i in range(N_DEV):
            d = i + 1
            xcp[i].wait()
            if d < N_DEV:
                xb_ref[d] = xf_ref[i % 4].astype(jnp.float8_e4m3fn)
                if do_comm:
                    rdma = pltpu.make_async_remote_copy(
                        src_ref=xb_ref.at[d],
                        dst_ref=xg_ref.at[:, pl.ds(d * k_loc, k_loc)],
                        send_sem=send_sems.at[d - 1],
                        recv_sem=recv_sems.at[d - 1],
                        device_id=((me + d) % N_DEV,),
                        device_id_type=pl.DeviceIdType.MESH,
                    )
                    rdma.start()
                    rdmas.append(rdma)
            else:
                xg_ref[:, 0:k_loc] = xf_ref[i % 4].astype(jnp.float8_e4m3fn)
            if i + 4 < N_DEV:
                xcp.append(start_xdma(i + 4, i % 4))

        for c in range(n_chunk):
            lo, hi = c * spc, (c + 1) * spc
            for p in range(lo, hi):
                wcp[p].wait()
                if _ABLATE != "nowcast":
                    wg_ref[p * k_loc:(p + 1) * k_loc, :] = (
                        wf_ref[p % 4].astype(_MM_DTYPE)
                    )
                if p + 4 < N_DEV:
                    wcp.append(start_wdma(p + 4, p % 4))
            if do_comm:
                for d in range(max(lo, 1), hi):
                    rdmas[d - 1].wait_recv()
            g = _mm(
                xg_ref[:, lo * k_loc:hi * k_loc],
                wg_ref[lo * k_loc:hi * k_loc, :],
            )
            if c == 0:
                acc_ref[...] = g
            else:
                acc_ref[...] += g

        ocp = pltpu.make_async_copy(acc_ref, acc_hbm, odma_sem)
        ocp.start()
        if do_comm:
            for d in range(1, N_DEV):
                rdmas[d - 1].wait_send()
        ocp.wait()

    acc = pl.pallas_call(
        body,
        out_shape=jax.ShapeDtypeStruct((mb, n_out), jnp.float32),
        in_specs=[
            pl.BlockSpec(memory_space=pltpu.MemorySpace.HBM),
            pl.BlockSpec(memory_space=pltpu.MemorySpace.HBM),
        ],
        out_specs=pl.BlockSpec(memory_space=pltpu.MemorySpace.HBM),
        scratch_shapes=[
            pltpu.VMEM((4, mb, k_loc), jnp.float32),
            pltpu.VMEM((N_DEV, mb, k_loc), jnp.float8_e4m3fn),
            pltpu.VMEM((mb, k_glob), jnp.float8_e4m3fn),
            pltpu.VMEM((4, k_loc, n_out), jnp.float32),
            pltpu.VMEM((k_glob, n_out), _MM_DTYPE),
            pltpu.VMEM((mb, n_out), jnp.float32),
            pltpu.SemaphoreType.DMA((N_DEV - 1,)),
            pltpu.SemaphoreType.DMA((N_DEV - 1,)),
            pltpu.SemaphoreType.DMA((4,)),
            pltpu.SemaphoreType.DMA((4,)),
            pltpu.SemaphoreType.DMA,
        ],
        compiler_params=pltpu.CompilerParams(
            collective_id=None if not do_comm else 0,
            vmem_limit_bytes=100 * 1024 * 1024,
        ),
        interpret=pltpu.InterpretParams() if _INTERPRET else False,
    )(x, w_mat)

    hb = mb // 2

    def epilogue(acc_hbm, sx_ref, sw_ref, out_ref, af_ref, sems):
        cps = []
        for h in range(2):
            cp = pltpu.make_async_copy(
                acc_hbm.at[pl.ds(h * hb, hb), :], af_ref.at[h], sems.at[h]
            )
            cp.start()
            cps.append(cp)
        s = sx_ref[0] * sw_ref[0]
        for h in range(2):
            cps[h].wait()
            y = af_ref[h] * s
            out_ref[h * hb:(h + 1) * hb, :] = (
                y * jax.nn.sigmoid(jnp.clip(y, -60.0, 60.0))
            )

    return pl.pallas_call(
        epilogue,
        out_shape=jax.ShapeDtypeStruct((mb, n_out), jnp.float32),
        in_specs=[
            pl.BlockSpec(memory_space=pltpu.MemorySpace.HBM),
            pl.BlockSpec(memory_space=pltpu.SMEM),
            pl.BlockSpec(memory_space=pltpu.SMEM),
        ],
        out_specs=pl.BlockSpec(memory_space=pltpu.VMEM),
        scratch_shapes=[
            pltpu.VMEM((2, hb, n_out), jnp.float32),
            pltpu.SemaphoreType.DMA((2,)),
        ],
        interpret=pltpu.InterpretParams() if _INTERPRET else False,
    )(acc, scale_x, scale_w)


# device time: 29338 ns/iter; 1.0693x vs baseline; 1.0693x over previous
import os

import jax
import jax.numpy as jnp
from jax import lax
from jax.experimental import pallas as pl
from jax.experimental.pallas import tpu as pltpu

N_DEV = 8
_INTERPRET = os.environ.get("KERNEL_INTERPRET", "0") == "1"
_MM_DTYPE = jnp.bfloat16 if os.environ.get("KERNEL_MM_BF16") == "1" else jnp.float8_e4m3fn
_ABLATE = os.environ.get("KERNEL_ABLATE", "")


def _mm(a, b):
    return lax.dot_general(
        a, b, (((1,), (0,)), ((), ())), preferred_element_type=jnp.float32
    )


def kernel(x, w_mat, scale_x, scale_w):
    m_glob, k_loc = x.shape
    k_glob, n_out = w_mat.shape
    mb = m_glob // N_DEV
    n_chunk = 4
    spc = N_DEV // n_chunk
    do_comm = _ABLATE != "nocomm"

    def body(x_hbm, w_hbm, acc_hbm,
             xf_ref, xb_ref, xg_ref, wf_ref, wg_ref, acc_ref,
             send_sems, recv_sems, wdma_sems, xdma_sems, odma_sem):
        me = lax.axis_index("i")

        def kblk(p):
            return (me - p) % N_DEV

        def start_wdma(step, slot):
            cp = pltpu.make_async_copy(
                w_hbm.at[pl.ds(kblk(step) * k_loc, k_loc), :],
                wf_ref.at[slot],
                wdma_sems.at[slot],
            )
            cp.start()
            return cp

        def start_xdma(i, slot):
            cp = pltpu.make_async_copy(
                x_hbm.at[pl.ds(((me + i + 1) % N_DEV) * mb, mb), :],
                xf_ref.at[slot],
                xdma_sems.at[slot],
            )
            cp.start()
            return cp

        xcp = [start_xdma(i, i) for i in range(N_DEV)]

        if do_comm:
            barrier = pltpu.get_barrier_semaphore()
            for d in range(1, N_DEV):
                pl.semaphore_signal(
                    barrier, inc=1,
                    device_id=((me + d) % N_DEV,),
                    device_id_type=pl.DeviceIdType.MESH,
                )
            pl.semaphore_wait(barrier, N_DEV - 1)

        rdmas = []
        for i in range(N_DEV):
            d = i + 1
            xcp[i].wait()
            if d < N_DEV:
                xb_ref[d] = xf_ref[i].astype(jnp.float8_e4m3fn)
                if do_comm:
                    rdma = pltpu.make_async_remote_copy(
                        src_ref=xb_ref.at[d],
                        dst_ref=xg_ref.at[:, pl.ds(d * k_loc, k_loc)],
                        send_sem=send_sems.at[d - 1],
                        recv_sem=recv_sems.at[d - 1],
                        device_id=((me + d) % N_DEV,),
                        device_id_type=pl.DeviceIdType.MESH,
                    )
                    rdma.start()
                    rdmas.append(rdma)
            else:
                xg_ref[:, 0:k_loc] = xf_ref[i].astype(jnp.float8_e4m3fn)

        wcp = [start_wdma(p, p) for p in range(4)]

        for c in range(n_chunk):
            lo, hi = c * spc, (c + 1) * spc
            for p in range(lo, hi):
                wcp[p].wait()
                if _ABLATE != "nowcast":
                    wg_ref[p * k_loc:(p + 1) * k_loc, :] = (
                        wf_ref[p % 4].astype(_MM_DTYPE)
                    )
                if p + 4 < N_DEV:
                    wcp.append(start_wdma(p + 4, p % 4))
            if do_comm:
                for d in range(max(lo, 1), hi):
                    rdmas[d - 1].wait_recv()
            g = _mm(
                xg_ref[:, lo * k_loc:hi * k_loc],
                wg_ref[lo * k_loc:hi * k_loc, :],
            )
            if c == 0:
                acc_ref[...] = g
            else:
                acc_ref[...] += g

        ocp = pltpu.make_async_copy(acc_ref, acc_hbm, odma_sem)
        ocp.start()
        if do_comm:
            for d in range(1, N_DEV):
                rdmas[d - 1].wait_send()
        ocp.wait()

    acc = pl.pallas_call(
        body,
        out_shape=jax.ShapeDtypeStruct((mb, n_out), jnp.float32),
        in_specs=[
            pl.BlockSpec(memory_space=pltpu.MemorySpace.HBM),
            pl.BlockSpec(memory_space=pltpu.MemorySpace.HBM),
        ],
        out_specs=pl.BlockSpec(memory_space=pltpu.MemorySpace.HBM),
        scratch_shapes=[
            pltpu.VMEM((N_DEV, mb, k_loc), jnp.float32),
            pltpu.VMEM((N_DEV, mb, k_loc), jnp.float8_e4m3fn),
            pltpu.VMEM((mb, k_glob), jnp.float8_e4m3fn),
            pltpu.VMEM((4, k_loc, n_out), jnp.float32),
            pltpu.VMEM((k_glob, n_out), _MM_DTYPE),
            pltpu.VMEM((mb, n_out), jnp.float32),
            pltpu.SemaphoreType.DMA((N_DEV - 1,)),
            pltpu.SemaphoreType.DMA((N_DEV - 1,)),
            pltpu.SemaphoreType.DMA((4,)),
            pltpu.SemaphoreType.DMA((N_DEV,)),
            pltpu.SemaphoreType.DMA,
        ],
        compiler_params=pltpu.CompilerParams(
            collective_id=None if not do_comm else 0,
            vmem_limit_bytes=100 * 1024 * 1024,
        ),
        interpret=pltpu.InterpretParams() if _INTERPRET else False,
    )(x, w_mat)

    def epilogue(acc_ref, sx_ref, sw_ref, out_ref):
        y = acc_ref[...] * (sx_ref[0] * sw_ref[0])
        out_ref[...] = y * jax.nn.sigmoid(jnp.clip(y, -60.0, 60.0))

    return pl.pallas_call(
        epilogue,
        out_shape=jax.ShapeDtypeStruct((mb, n_out), jnp.float32),
        in_specs=[
            pl.BlockSpec(memory_space=pltpu.VMEM),
            pl.BlockSpec(memory_space=pltpu.SMEM),
            pl.BlockSpec(memory_space=pltpu.SMEM),
        ],
        out_specs=pl.BlockSpec(memory_space=pltpu.VMEM),
        interpret=pltpu.InterpretParams() if _INTERPRET else False,
    )(acc, scale_x, scale_w)


# device time: 28334 ns/iter; 1.1072x vs baseline; 1.0354x over previous
import os

import jax
import jax.numpy as jnp
from jax import lax
from jax.experimental import pallas as pl
from jax.experimental.pallas import tpu as pltpu

N_DEV = 8
_INTERPRET = os.environ.get("KERNEL_INTERPRET", "0") == "1"
_MM_DTYPE = jnp.bfloat16 if os.environ.get("KERNEL_MM_BF16") == "1" else jnp.float8_e4m3fn
_ABLATE = os.environ.get("KERNEL_ABLATE", "")


def _mm(a, b):
    return lax.dot_general(
        a, b, (((1,), (0,)), ((), ())), preferred_element_type=jnp.float32
    )


def kernel(x, w_mat, scale_x, scale_w):
    m_glob, k_loc = x.shape
    k_glob, n_out = w_mat.shape
    mb = m_glob // N_DEV
    do_comm = _ABLATE != "nocomm"

    def body(x_hbm, w_hbm, acc_hbm,
             xf_ref, xb_ref, xg_ref, wf_ref, acc_ref, ab_ref,
             send_sems, recv_sems, wdma_sems, xdma_sems, odma_sem):
        me = lax.axis_index("i")

        def kblk(p):
            return (me - p) % N_DEV

        def start_wdma(step, slot):
            cp = pltpu.make_async_copy(
                w_hbm.at[pl.ds(kblk(step) * k_loc, k_loc), :],
                wf_ref.at[slot],
                wdma_sems.at[slot],
            )
            cp.start()
            return cp

        def start_xdma(i, slot):
            cp = pltpu.make_async_copy(
                x_hbm.at[pl.ds(((me + i + 1) % N_DEV) * mb, mb), :],
                xf_ref.at[slot],
                xdma_sems.at[slot],
            )
            cp.start()
            return cp

        xcp = [start_xdma(i, i) for i in range(N_DEV)]

        if do_comm:
            barrier = pltpu.get_barrier_semaphore()
            for d in range(1, N_DEV):
                pl.semaphore_signal(
                    barrier, inc=1,
                    device_id=((me + d) % N_DEV,),
                    device_id_type=pl.DeviceIdType.MESH,
                )
            pl.semaphore_wait(barrier, N_DEV - 1)

        rdmas = []
        for i in range(N_DEV):
            d = i + 1
            xcp[i].wait()
            if d < N_DEV:
                xb_ref[d] = xf_ref[i].astype(jnp.float8_e4m3fn)
                if do_comm:
                    rdma = pltpu.make_async_remote_copy(
                        src_ref=xb_ref.at[d],
                        dst_ref=xg_ref.at[:, pl.ds(d * k_loc, k_loc)],
                        send_sem=send_sems.at[d - 1],
                        recv_sem=recv_sems.at[d - 1],
                        device_id=((me + d) % N_DEV,),
                        device_id_type=pl.DeviceIdType.MESH,
                    )
                    rdma.start()
                    rdmas.append(rdma)
            else:
                xg_ref[:, 0:k_loc] = xf_ref[i].astype(jnp.float8_e4m3fn)

        wcp = [start_wdma(p, p) for p in range(6)]

        for p in range(N_DEV):
            wcp[p].wait()
            if do_comm and p >= 1:
                rdmas[p - 1].wait_recv()
            g = _mm(
                xg_ref[:, p * k_loc:(p + 1) * k_loc],
                wf_ref[p % 6].astype(_MM_DTYPE),
            )
            if p + 6 < N_DEV:
                wcp.append(start_wdma(p + 6, p % 6))
            if p == 0:
                acc_ref[...] = g
            else:
                acc_ref[...] += g

        ab_ref[...] = acc_ref[...].astype(jnp.bfloat16)
        ocp = pltpu.make_async_copy(ab_ref, acc_hbm, odma_sem)
        ocp.start()
        if do_comm:
            for d in range(1, N_DEV):
                rdmas[d - 1].wait_send()
        ocp.wait()

    acc = pl.pallas_call(
        body,
        out_shape=jax.ShapeDtypeStruct((mb, n_out), jnp.bfloat16),
        in_specs=[
            pl.BlockSpec(memory_space=pltpu.MemorySpace.HBM),
            pl.BlockSpec(memory_space=pltpu.MemorySpace.HBM),
        ],
        out_specs=pl.BlockSpec(memory_space=pltpu.MemorySpace.HBM),
        scratch_shapes=[
            pltpu.VMEM((N_DEV, mb, k_loc), jnp.float32),
            pltpu.VMEM((N_DEV, mb, k_loc), jnp.float8_e4m3fn),
            pltpu.VMEM((mb, k_glob), jnp.float8_e4m3fn),
            pltpu.VMEM((6, k_loc, n_out), jnp.float32),
            pltpu.VMEM((mb, n_out), jnp.float32),
            pltpu.VMEM((mb, n_out), jnp.bfloat16),
            pltpu.SemaphoreType.DMA((N_DEV - 1,)),
            pltpu.SemaphoreType.DMA((N_DEV - 1,)),
            pltpu.SemaphoreType.DMA((6,)),
            pltpu.SemaphoreType.DMA((N_DEV,)),
            pltpu.SemaphoreType.DMA,
        ],
        compiler_params=pltpu.CompilerParams(
            collective_id=None if not do_comm else 0,
            vmem_limit_bytes=100 * 1024 * 1024,
        ),
        interpret=pltpu.InterpretParams() if _INTERPRET else False,
    )(x, w_mat)

    def epilogue(acc_ref, sx_ref, sw_ref, out_ref):
        y = acc_ref[...].astype(jnp.float32) * (sx_ref[0] * sw_ref[0])
        out_ref[...] = y * jax.nn.sigmoid(jnp.clip(y, -60.0, 60.0))

    return pl.pallas_call(
        epilogue,
        out_shape=jax.ShapeDtypeStruct((mb, n_out), jnp.float32),
        in_specs=[
            pl.BlockSpec(memory_space=pltpu.VMEM),
            pl.BlockSpec(memory_space=pltpu.SMEM),
            pl.BlockSpec(memory_space=pltpu.SMEM),
        ],
        out_specs=pl.BlockSpec(memory_space=pltpu.VMEM),
        interpret=pltpu.InterpretParams() if _INTERPRET else False,
    )(acc, scale_x, scale_w)
